# baseline (speedup 1.0000x reference)
import functools

import jax
import jax.numpy as jnp
from jax import lax
from jax.experimental import pallas as pl
from jax.experimental.pallas import tpu as pltpu
from jax.experimental.pallas import tpu_sc as plsc

N_ENT = 100000
N_REL = 1000
DIM = 128
B = 16384
BETA = 1.0
EPS = 1e-23
REG_DELTA = 0.05
REG_MIN = 0.0005
REG_REL = 0.0005

_NC = 2
_NS = 16
_NW = _NC * _NS
_CHUNK = 128
_NSPLIT = 4


def _sc_gather_body(nrows, n_groups, refs):
    (groups, idx_all, bufs, isem, gsems, ssems) = refs
    _NCHUNK = nrows // _NW // _CHUNK
    wid = lax.axis_index("s") * _NC + lax.axis_index("c")
    base = wid * (nrows // _NW)

    idx_copies = {}
    for g, (idx_hbm, _) in enumerate(groups):
        for c in range(_NCHUNK):
            j = g * _NCHUNK + c
            idx_copies[j] = pltpu.async_copy(
                idx_hbm.at[pl.ds(base + c * _CHUNK, _CHUNK)],
                idx_all.at[j], isem)

    units = []
    for c in range(_NCHUNK):
        for g, (_, pairs) in enumerate(groups):
            for table, out in pairs:
                units.append((g * _NCHUNK + c, table, out, base + c * _CHUNK))

    nbuf = len(gsems)
    gathers = [None] * nbuf
    stores = [None] * nbuf
    idx_done = set()

    def start_gather(k):
        slot = k % nbuf
        j, table, _, _ = units[k]
        if stores[slot] is not None:
            stores[slot].wait()
        if j not in idx_done:
            idx_copies[j].wait()
            idx_done.add(j)
        gathers[slot] = pltpu.async_copy(
            table.at[idx_all.at[j]], bufs.at[slot], gsems[slot])

    start_gather(0)
    for k in range(len(units)):
        if k + 1 < len(units):
            start_gather(k + 1)
        slot = k % nbuf
        _, _, out, row0 = units[k]
        gathers[slot].wait()
        stores[slot] = pltpu.async_copy(
            bufs.at[slot], out.at[pl.ds(row0, _CHUNK)], ssems[slot])
    for st in stores:
        if st is not None:
            st.wait()


_NBUF = 6
_REL_ON_SC = 0


def _make_sc_gather(nrows):
    row = jax.ShapeDtypeStruct((nrows, DIM), jnp.float32)
    nchunk = nrows // _NW // _CHUNK
    n_ent_groups = 4
    n_out = 8 + _REL_ON_SC

    def body(*refs):
        n_tab = 2 + _REL_ON_SC
        tabs = refs[:n_tab]
        idxs = refs[n_tab:n_tab + 4 + (1 if _REL_ON_SC else 0)]
        outs = refs[n_tab + len(idxs):n_tab + len(idxs) + n_out]
        idx_all, bufs, isem, gsems, ssems = refs[n_tab + len(idxs) + n_out:]
        min_e, delta_e = tabs[0], tabs[1]
        groups = [
            (idxs[0], ((min_e, outs[0]), (delta_e, outs[1]))),
            (idxs[1], ((min_e, outs[2]), (delta_e, outs[3]))),
            (idxs[2], ((min_e, outs[4]), (delta_e, outs[5]))),
            (idxs[3], ((min_e, outs[6]), (delta_e, outs[7]))),
        ]
        if _REL_ON_SC:
            groups.append((idxs[4], tuple(
                (tabs[2 + k], outs[8 + k]) for k in range(_REL_ON_SC))))
        _sc_gather_body(nrows, len(groups),
                        (groups, idx_all, bufs, isem, gsems, ssems))

    return pl.kernel(
        body,
        out_type=[row] * n_out,
        mesh=plsc.VectorSubcoreMesh(core_axis_name="c", subcore_axis_name="s"),
        scratch_types=[
            pltpu.VMEM((5 * nchunk, _CHUNK), jnp.int32),
            pltpu.VMEM((_NBUF, _CHUNK, DIM), jnp.float32),
            pltpu.SemaphoreType.DMA,
            [pltpu.SemaphoreType.DMA] * _NBUF,
            [pltpu.SemaphoreType.DMA] * _NBUF,
        ],
    )


def _log1p(x):
    u = 1.0 + x
    d = u - 1.0
    return jnp.where(d == 0.0, x, jnp.log(u) * (x / d))


def _logaddexp(a, b):
    mx = jnp.maximum(a, b)
    return mx + _log1p(jnp.exp(-jnp.abs(a - b)))


def _softplus(x):
    return jnp.maximum(x, 0.0) + _log1p(jnp.exp(-jnp.abs(x)))


def _log_volume(bmin, bmax):
    return jnp.sum(jnp.log(_softplus((bmax - bmin) / BETA) * BETA + EPS),
                   axis=1, keepdims=True)


def _pred(h_min, h_max, t_min, t_max):
    meet_min = BETA * _logaddexp(h_min / BETA, t_min / BETA)
    meet_max = -BETA * _logaddexp(-h_max / BETA, -t_max / BETA)
    log_int = _log_volume(meet_min, meet_max)
    log_tail = _log_volume(t_min, t_max)
    return jnp.exp(jnp.minimum(log_int - log_tail, 0.0))


def _rownorm(x):
    return jnp.sqrt(jnp.sum(x * x, axis=1, keepdims=True))


_BB = 512


_NRELP = 1024


def _tc_loss_body(nb, *args):
    k = _REL_ON_SC
    (mh, dh, mt, dt, mnh, dnh, mnt, dnt) = args[:8]
    screl = args[8:8 + k]
    rest = args[8 + k:]
    if k < 4:
        rel_hi, rel_lo, rv, conf, out_ref, acc_ref = rest
    else:
        conf, out_ref, acc_ref = rest
    i = pl.program_id(0)

    @pl.when(i == 0)
    def _():
        acc_ref[0] = 0.0

    relrows = [r[...] for r in screl]
    if k < 4:
        cols = jax.lax.broadcasted_iota(jnp.int32, (_BB, _NRELP), 1)
        onehot = (cols == rv[...]).astype(jnp.bfloat16)
        rows = (jnp.dot(onehot, rel_hi[...], preferred_element_type=jnp.float32)
                + jnp.dot(onehot, rel_lo[...], preferred_element_type=jnp.float32))
        for j in range(4 - k):
            relrows.append(rows[:, j * DIM:(j + 1) * DIM])
    rth, rsh, rtt, rst = relrows

    sc_h = jnp.exp(rsh)
    sc_t = jnp.exp(rst)
    edh = jnp.exp(dh[...])
    edt = jnp.exp(dt[...])

    h_min = mh[...] * sc_h + rth
    h_max = h_min + edh * sc_h
    t_min = mt[...] * sc_t + rtt
    t_max = t_min + edt * sc_t
    pos = _pred(h_min, h_max, t_min, t_max)

    nh_min = mnh[...] * sc_h + rth
    nh_max = nh_min + jnp.exp(dnh[...]) * sc_h
    nt_min = mnt[...] * sc_t + rtt
    nt_max = nt_min + jnp.exp(dnt[...]) * sc_t
    neg = _pred(nh_min, nh_max, nt_min, nt_max)

    se = (pos - conf[...]) ** 2 + neg * neg
    reg = (REG_DELTA * (_rownorm(edh) + _rownorm(edt))
           + REG_MIN * (_rownorm(mh[...]) + _rownorm(mt[...]))
           + REG_REL * (_rownorm(jnp.exp(rth)) + _rownorm(jnp.exp(rtt)))
           + REG_REL * (_rownorm(sc_h) + _rownorm(sc_t)))
    acc_ref[0] += jnp.sum(se) + jnp.sum(reg)

    @pl.when(i == nb - 1)
    def _():
        out_ref[...] = jnp.full((1, 1), acc_ref[0], jnp.float32)


def _make_tc_loss(nrows):
    nb = nrows // _BB
    k = _REL_ON_SC
    row_spec = pl.BlockSpec((_BB, DIM), lambda i: (i, 0))
    rel_spec = pl.BlockSpec((_NRELP, (4 - k) * DIM), lambda i: (0, 0))
    col_spec = pl.BlockSpec((_BB, 1), lambda i: (i, 0))
    in_specs = [row_spec] * (8 + k)
    if k < 4:
        in_specs += [rel_spec] * 2 + [col_spec]
    in_specs += [col_spec]
    return pl.pallas_call(
        functools.partial(_tc_loss_body, nb),
        grid=(nb,),
        in_specs=in_specs,
        out_specs=pl.BlockSpec((1, 1), lambda i: (0, 0)),
        out_shape=jax.ShapeDtypeStruct((1, 1), jnp.float32),
        scratch_shapes=[pltpu.SMEM((1,), jnp.float32)],
    )


def kernel(ids, negative_samples, confidence, min_embedding, delta_embedding,
           rel_trans_for_head, rel_scale_for_head, rel_trans_for_tail,
           rel_scale_for_tail):
    ids = ids.astype(jnp.int32)
    neg = negative_samples.astype(jnp.int32)
    h = ids[:, 0]
    r = ids[:, 1]
    t = ids[:, 2]
    nh = neg[:, 0]
    nt = neg[:, 2]
    n = B // _NSPLIT
    k = _REL_ON_SC
    sc_fn = _make_sc_gather(n)
    tc_fn = _make_tc_loss(n)
    conf2d = confidence.reshape(B, 1)
    rel_tables = [rel_trans_for_head, rel_scale_for_head,
                  rel_trans_for_tail, rel_scale_for_tail]
    extra = []
    if k < 4:
        r2d = r.reshape(B, 1)
        rel_cat = jnp.concatenate(rel_tables[k:], axis=1)
        rel_cat = jnp.pad(rel_cat, ((0, _NRELP - N_REL), (0, 0)))
        rel_hi = rel_cat.astype(jnp.bfloat16)
        rel_lo = (rel_cat - rel_hi.astype(jnp.float32)).astype(jnp.bfloat16)
    partials = []
    for s in range(_NSPLIT):
        sl = slice(s * n, (s + 1) * n)
        sc_args = ([min_embedding, delta_embedding] + rel_tables[:k]
                   + [h[sl], t[sl], nh[sl], nt[sl]]
                   + ([r[sl]] if k else []))
        gathered = sc_fn(*sc_args)
        tc_args = list(gathered)
        if k < 4:
            tc_args += [rel_hi, rel_lo, r2d[sl]]
        tc_args.append(conf2d[sl])
        partials.append(tc_fn(*tc_args))
    total = partials[0]
    for p in partials[1:]:
        total = total + p
    return (total * (1.0 / B)).reshape(())


# speedup vs baseline: 1.1536x; 1.1536x over previous
import functools

import jax
import jax.numpy as jnp
from jax import lax
from jax.experimental import pallas as pl
from jax.experimental.pallas import tpu as pltpu
from jax.experimental.pallas import tpu_sc as plsc

N_ENT = 100000
N_REL = 1000
DIM = 128
B = 16384
BETA = 1.0
EPS = 1e-23
REG_DELTA = 0.05
REG_MIN = 0.0005
REG_REL = 0.0005

_NC = 2
_NS = 16
_NW = _NC * _NS
_CHUNK = 128
_NSPLIT = 4


def _sc_gather_body(nrows, n_groups, refs):
    (groups, idx_all, bufs, isem, gsems, ssems) = refs
    _NCHUNK = nrows // _NW // _CHUNK
    wid = lax.axis_index("s") * _NC + lax.axis_index("c")
    base = wid * (nrows // _NW)

    idx_copies = {}
    for g, (idx_hbm, _) in enumerate(groups):
        for c in range(_NCHUNK):
            j = g * _NCHUNK + c
            idx_copies[j] = pltpu.async_copy(
                idx_hbm.at[pl.ds(base + c * _CHUNK, _CHUNK)],
                idx_all.at[j], isem)

    units = []
    for c in range(_NCHUNK):
        for g, (_, pairs) in enumerate(groups):
            for table, out in pairs:
                units.append((g * _NCHUNK + c, table, out, base + c * _CHUNK))

    nbuf = len(gsems)
    gathers = [None] * nbuf
    stores = [None] * nbuf
    idx_done = set()

    def start_gather(k):
        slot = k % nbuf
        j, table, _, _ = units[k]
        if stores[slot] is not None:
            stores[slot].wait()
        if j not in idx_done:
            idx_copies[j].wait()
            idx_done.add(j)
        gathers[slot] = pltpu.async_copy(
            table.at[idx_all.at[j]], bufs.at[slot], gsems[slot])

    start_gather(0)
    for k in range(len(units)):
        if k + 1 < len(units):
            start_gather(k + 1)
        slot = k % nbuf
        _, _, out, row0 = units[k]
        gathers[slot].wait()
        stores[slot] = pltpu.async_copy(
            bufs.at[slot], out.at[pl.ds(row0, _CHUNK)], ssems[slot])
    for st in stores:
        if st is not None:
            st.wait()


_NBUF = 6
_REL_ON_SC = 4


def _make_sc_gather(nrows):
    row = jax.ShapeDtypeStruct((nrows, DIM), jnp.float32)
    nchunk = nrows // _NW // _CHUNK
    n_ent_groups = 4
    n_out = 8 + _REL_ON_SC

    def body(*refs):
        n_tab = 2 + _REL_ON_SC
        tabs = refs[:n_tab]
        idxs = refs[n_tab:n_tab + 4 + (1 if _REL_ON_SC else 0)]
        outs = refs[n_tab + len(idxs):n_tab + len(idxs) + n_out]
        idx_all, bufs, isem, gsems, ssems = refs[n_tab + len(idxs) + n_out:]
        min_e, delta_e = tabs[0], tabs[1]
        groups = [
            (idxs[0], ((min_e, outs[0]), (delta_e, outs[1]))),
            (idxs[1], ((min_e, outs[2]), (delta_e, outs[3]))),
            (idxs[2], ((min_e, outs[4]), (delta_e, outs[5]))),
            (idxs[3], ((min_e, outs[6]), (delta_e, outs[7]))),
        ]
        if _REL_ON_SC:
            groups.append((idxs[4], tuple(
                (tabs[2 + k], outs[8 + k]) for k in range(_REL_ON_SC))))
        _sc_gather_body(nrows, len(groups),
                        (groups, idx_all, bufs, isem, gsems, ssems))

    return pl.kernel(
        body,
        out_type=[row] * n_out,
        mesh=plsc.VectorSubcoreMesh(core_axis_name="c", subcore_axis_name="s"),
        scratch_types=[
            pltpu.VMEM((5 * nchunk, _CHUNK), jnp.int32),
            pltpu.VMEM((_NBUF, _CHUNK, DIM), jnp.float32),
            pltpu.SemaphoreType.DMA,
            [pltpu.SemaphoreType.DMA] * _NBUF,
            [pltpu.SemaphoreType.DMA] * _NBUF,
        ],
    )


def _log1p(x):
    u = 1.0 + x
    d = u - 1.0
    return jnp.where(d == 0.0, x, jnp.log(u) * (x / d))


def _logaddexp(a, b):
    mx = jnp.maximum(a, b)
    return mx + _log1p(jnp.exp(-jnp.abs(a - b)))


def _softplus(x):
    return jnp.maximum(x, 0.0) + _log1p(jnp.exp(-jnp.abs(x)))


def _log_volume(bmin, bmax):
    return jnp.sum(jnp.log(_softplus((bmax - bmin) / BETA) * BETA + EPS),
                   axis=1, keepdims=True)


def _pred(h_min, h_max, t_min, t_max):
    meet_min = BETA * _logaddexp(h_min / BETA, t_min / BETA)
    meet_max = -BETA * _logaddexp(-h_max / BETA, -t_max / BETA)
    log_int = _log_volume(meet_min, meet_max)
    log_tail = _log_volume(t_min, t_max)
    return jnp.exp(jnp.minimum(log_int - log_tail, 0.0))


def _rownorm(x):
    return jnp.sqrt(jnp.sum(x * x, axis=1, keepdims=True))


_BB = 512


_NRELP = 1024


def _tc_loss_body(nb, *args):
    k = _REL_ON_SC
    (mh, dh, mt, dt, mnh, dnh, mnt, dnt) = args[:8]
    screl = args[8:8 + k]
    rest = args[8 + k:]
    if k < 4:
        rel_hi, rel_lo, rv, conf, out_ref, acc_ref = rest
    else:
        conf, out_ref, acc_ref = rest
    i = pl.program_id(0)

    @pl.when(i == 0)
    def _():
        acc_ref[0] = 0.0

    relrows = [r[...] for r in screl]
    if k < 4:
        cols = jax.lax.broadcasted_iota(jnp.int32, (_BB, _NRELP), 1)
        onehot = (cols == rv[...]).astype(jnp.bfloat16)
        rows = (jnp.dot(onehot, rel_hi[...], preferred_element_type=jnp.float32)
                + jnp.dot(onehot, rel_lo[...], preferred_element_type=jnp.float32))
        for j in range(4 - k):
            relrows.append(rows[:, j * DIM:(j + 1) * DIM])
    rth, rsh, rtt, rst = relrows

    sc_h = jnp.exp(rsh)
    sc_t = jnp.exp(rst)
    edh = jnp.exp(dh[...])
    edt = jnp.exp(dt[...])

    h_min = mh[...] * sc_h + rth
    h_max = h_min + edh * sc_h
    t_min = mt[...] * sc_t + rtt
    t_max = t_min + edt * sc_t
    pos = _pred(h_min, h_max, t_min, t_max)

    nh_min = mnh[...] * sc_h + rth
    nh_max = nh_min + jnp.exp(dnh[...]) * sc_h
    nt_min = mnt[...] * sc_t + rtt
    nt_max = nt_min + jnp.exp(dnt[...]) * sc_t
    neg = _pred(nh_min, nh_max, nt_min, nt_max)

    se = (pos - conf[...]) ** 2 + neg * neg
    reg = (REG_DELTA * (_rownorm(edh) + _rownorm(edt))
           + REG_MIN * (_rownorm(mh[...]) + _rownorm(mt[...]))
           + REG_REL * (_rownorm(jnp.exp(rth)) + _rownorm(jnp.exp(rtt)))
           + REG_REL * (_rownorm(sc_h) + _rownorm(sc_t)))
    acc_ref[0] += jnp.sum(se) + jnp.sum(reg)

    @pl.when(i == nb - 1)
    def _():
        out_ref[...] = jnp.full((1, 1), acc_ref[0], jnp.float32)


def _make_tc_loss(nrows):
    nb = nrows // _BB
    k = _REL_ON_SC
    row_spec = pl.BlockSpec((_BB, DIM), lambda i: (i, 0))
    rel_spec = pl.BlockSpec((_NRELP, (4 - k) * DIM), lambda i: (0, 0))
    col_spec = pl.BlockSpec((_BB, 1), lambda i: (i, 0))
    in_specs = [row_spec] * (8 + k)
    if k < 4:
        in_specs += [rel_spec] * 2 + [col_spec]
    in_specs += [col_spec]
    return pl.pallas_call(
        functools.partial(_tc_loss_body, nb),
        grid=(nb,),
        in_specs=in_specs,
        out_specs=pl.BlockSpec((1, 1), lambda i: (0, 0)),
        out_shape=jax.ShapeDtypeStruct((1, 1), jnp.float32),
        scratch_shapes=[pltpu.SMEM((1,), jnp.float32)],
    )


def kernel(ids, negative_samples, confidence, min_embedding, delta_embedding,
           rel_trans_for_head, rel_scale_for_head, rel_trans_for_tail,
           rel_scale_for_tail):
    ids = ids.astype(jnp.int32)
    neg = negative_samples.astype(jnp.int32)
    h = ids[:, 0]
    r = ids[:, 1]
    t = ids[:, 2]
    nh = neg[:, 0]
    nt = neg[:, 2]
    n = B // _NSPLIT
    k = _REL_ON_SC
    sc_fn = _make_sc_gather(n)
    tc_fn = _make_tc_loss(n)
    conf2d = confidence.reshape(B, 1)
    rel_tables = [rel_trans_for_head, rel_scale_for_head,
                  rel_trans_for_tail, rel_scale_for_tail]
    extra = []
    if k < 4:
        r2d = r.reshape(B, 1)
        rel_cat = jnp.concatenate(rel_tables[k:], axis=1)
        rel_cat = jnp.pad(rel_cat, ((0, _NRELP - N_REL), (0, 0)))
        rel_hi = rel_cat.astype(jnp.bfloat16)
        rel_lo = (rel_cat - rel_hi.astype(jnp.float32)).astype(jnp.bfloat16)
    partials = []
    for s in range(_NSPLIT):
        sl = slice(s * n, (s + 1) * n)
        sc_args = ([min_embedding, delta_embedding] + rel_tables[:k]
                   + [h[sl], t[sl], nh[sl], nt[sl]]
                   + ([r[sl]] if k else []))
        gathered = sc_fn(*sc_args)
        tc_args = list(gathered)
        if k < 4:
            tc_args += [rel_hi, rel_lo, r2d[sl]]
        tc_args.append(conf2d[sl])
        partials.append(tc_fn(*tc_args))
    total = partials[0]
    for p in partials[1:]:
        total = total + p
    return (total * (1.0 / B)).reshape(())
